# baseline (speedup 1.0000x reference)
import jax
import jax.numpy as jnp
from jax.experimental import pallas as pl
from jax.experimental.pallas import tpu as pltpu

S, D, H, DK, DV = 4096, 1024, 16, 64, 64
BM = 1024
BQ = 512
SCALE = 1.0 / (DK ** 0.5)
BF = jnp.bfloat16


def _qkv_kernel(x_ref, w_ref, q_ref, k_ref, v_ref):
    r = jnp.dot(x_ref[...].astype(BF), w_ref[...],
                preferred_element_type=jnp.float32)
    col = jax.lax.broadcasted_iota(jnp.int32, (BM, DV), 1)
    ones_col = jnp.where(col == 0, 1.0, 0.0).astype(BF)
    for h in range(H):
        q_ref[h] = r[:, h * DK:(h + 1) * DK].astype(BF)
        k_ref[h] = r[:, H * DK + h * DK:H * DK + (h + 1) * DK].astype(BF)
        v_ref[h] = jnp.concatenate(
            [r[:, 2 * H * DK + h * DV:2 * H * DK + (h + 1) * DV].astype(BF),
             ones_col], axis=1)


def _attn_kernel(q_ref, k_ref, v_ref, wp_ref, o_ref):
    c2 = SCALE * 1.4426950408889634
    outs = []
    for h in range(H):
        q = q_ref[h]
        k = k_ref[h]
        s = jax.lax.dot_general(q, k, (((1,), (1,)), ((), ())),
                                preferred_element_type=jnp.float32)
        m = jnp.max(s, axis=-1, keepdims=True)
        p = jnp.exp2((s - m) * c2).astype(BF)
        oe = jnp.dot(p, v_ref[h], preferred_element_type=jnp.float32)
        outs.append(oe[:, :DV] / oe[:, DV:DV + 1])
    concat = jnp.concatenate(outs, axis=-1)
    o_ref[...] = jnp.dot(concat.astype(BF), wp_ref[...],
                         preferred_element_type=jnp.float32)


def kernel(x, wq, wk, wv, w_proj):
    wq2 = wq.transpose(1, 0, 2).reshape(D, H * DK)
    wk2 = wk.transpose(1, 0, 2).reshape(D, H * DK)
    wv2 = wv.transpose(1, 0, 2).reshape(D, H * DV)
    w_all = jnp.concatenate([wq2, wk2, wv2], axis=1)

    w16 = w_all.astype(BF)
    wp16 = w_proj.astype(BF)

    q_all, k_all, v_all = pl.pallas_call(
        _qkv_kernel,
        grid=(S // BM,),
        in_specs=[
            pl.BlockSpec((BM, D), lambda i: (i, 0)),
            pl.BlockSpec((D, 3 * H * DK), lambda i: (0, 0)),
        ],
        out_specs=[
            pl.BlockSpec((H, BM, DK), lambda i: (0, i, 0)),
            pl.BlockSpec((H, BM, DK), lambda i: (0, i, 0)),
            pl.BlockSpec((H, BM, 2 * DV), lambda i: (0, i, 0)),
        ],
        out_shape=[
            jax.ShapeDtypeStruct((H, S, DK), BF),
            jax.ShapeDtypeStruct((H, S, DK), BF),
            jax.ShapeDtypeStruct((H, S, 2 * DV), BF),
        ],
        compiler_params=pltpu.CompilerParams(
            dimension_semantics=("parallel",)),
    )(x, w16)

    return pl.pallas_call(
        _attn_kernel,
        grid=(S // BQ,),
        in_specs=[
            pl.BlockSpec((H, BQ, DK), lambda i: (0, i, 0)),
            pl.BlockSpec((H, S, DK), lambda i: (0, 0, 0)),
            pl.BlockSpec((H, S, 2 * DV), lambda i: (0, 0, 0)),
            pl.BlockSpec((H * DV, D), lambda i: (0, 0)),
        ],
        out_specs=pl.BlockSpec((BQ, D), lambda i: (i, 0)),
        out_shape=jax.ShapeDtypeStruct((S, D), jnp.float32),
        compiler_params=pltpu.CompilerParams(
            dimension_semantics=("parallel",)),
    )(q_all, k_all, v_all, wp16)


# speedup vs baseline: 1.1346x; 1.1346x over previous
import jax
import jax.numpy as jnp
from jax.experimental import pallas as pl
from jax.experimental.pallas import tpu as pltpu

S, D, H, DK, DV = 4096, 1024, 16, 64, 64
BM = 1024
BQ = 256
SCALE = 1.0 / (DK ** 0.5)
BF = jnp.bfloat16


def _qkv_kernel(x_ref, w_ref, q_ref, k_ref, v_ref):
    r = jnp.dot(x_ref[...].astype(BF), w_ref[...],
                preferred_element_type=jnp.float32)
    col = jax.lax.broadcasted_iota(jnp.int32, (BM, DV), 1)
    ones_col = jnp.where(col == 0, 1.0, 0.0).astype(BF)
    for h in range(H):
        q_ref[h] = r[:, h * DK:(h + 1) * DK].astype(BF)
        k_ref[h] = r[:, H * DK + h * DK:H * DK + (h + 1) * DK].astype(BF)
        v_ref[h] = jnp.concatenate(
            [r[:, 2 * H * DK + h * DV:2 * H * DK + (h + 1) * DV].astype(BF),
             ones_col], axis=1)


def _attn_kernel(q_ref, k_ref, v_ref, wp_ref, o_ref):
    c2 = SCALE * 1.4426950408889634
    outs = []
    for h in range(H):
        q = q_ref[h]
        k = k_ref[h]
        s = jax.lax.dot_general(q, k, (((1,), (1,)), ((), ())),
                                preferred_element_type=jnp.float32)
        m = jnp.max(s, axis=-1, keepdims=True)
        p = jnp.exp2((s - m) * c2).astype(BF)
        oe = jnp.dot(p, v_ref[h], preferred_element_type=jnp.float32)
        outs.append(oe[:, :DV] / oe[:, DV:DV + 1])
    concat = jnp.concatenate(outs, axis=-1)
    o_ref[...] = jnp.dot(concat.astype(BF), wp_ref[...],
                         preferred_element_type=jnp.float32)


def kernel(x, wq, wk, wv, w_proj):
    wq2 = wq.transpose(1, 0, 2).reshape(D, H * DK)
    wk2 = wk.transpose(1, 0, 2).reshape(D, H * DK)
    wv2 = wv.transpose(1, 0, 2).reshape(D, H * DV)
    w_all = jnp.concatenate([wq2, wk2, wv2], axis=1)

    w16 = w_all.astype(BF)
    wp16 = w_proj.astype(BF)

    q_all, k_all, v_all = pl.pallas_call(
        _qkv_kernel,
        grid=(S // BM,),
        in_specs=[
            pl.BlockSpec((BM, D), lambda i: (i, 0)),
            pl.BlockSpec((D, 3 * H * DK), lambda i: (0, 0)),
        ],
        out_specs=[
            pl.BlockSpec((H, BM, DK), lambda i: (0, i, 0)),
            pl.BlockSpec((H, BM, DK), lambda i: (0, i, 0)),
            pl.BlockSpec((H, BM, 2 * DV), lambda i: (0, i, 0)),
        ],
        out_shape=[
            jax.ShapeDtypeStruct((H, S, DK), BF),
            jax.ShapeDtypeStruct((H, S, DK), BF),
            jax.ShapeDtypeStruct((H, S, 2 * DV), BF),
        ],
        compiler_params=pltpu.CompilerParams(
            dimension_semantics=("parallel",)),
    )(x, w16)

    return pl.pallas_call(
        _attn_kernel,
        grid=(S // BQ,),
        in_specs=[
            pl.BlockSpec((H, BQ, DK), lambda i: (0, i, 0)),
            pl.BlockSpec((H, S, DK), lambda i: (0, 0, 0)),
            pl.BlockSpec((H, S, 2 * DV), lambda i: (0, 0, 0)),
            pl.BlockSpec((H * DV, D), lambda i: (0, 0)),
        ],
        out_specs=pl.BlockSpec((BQ, D), lambda i: (i, 0)),
        out_shape=jax.ShapeDtypeStruct((S, D), jnp.float32),
        compiler_params=pltpu.CompilerParams(
            dimension_semantics=("parallel",)),
    )(q_all, k_all, v_all, wp16)
